# baseline (speedup 1.0000x reference)
import functools

import jax
import jax.numpy as jnp
from jax import lax
from jax.experimental import pallas as pl
from jax.experimental.pallas import tpu as pltpu
from jax.experimental.pallas import tpu_sc as plsc

B = 327680
D = 64
NUM_CORES = 2
NUM_SUBCORES = 16
NW = NUM_CORES * NUM_SUBCORES
ROWS_PER_W = B // NW
STREAM_ROWS = 128
CHUNK_STREAMS = 8
CHUNK_ROWS = STREAM_ROWS * CHUNK_STREAMS
N_CHUNKS = ROWS_PER_W // CHUNK_ROWS
IDX_ROWS_PER_W = ROWS_PER_W // STREAM_ROWS


def _gather_body(table_hbm, idx_hbm, out_hbm, idx_v, rows_v, sem):
    wid = lax.axis_index("s") * NUM_CORES + lax.axis_index("c")
    base = wid * ROWS_PER_W
    pltpu.sync_copy(idx_hbm.at[pl.ds(wid * IDX_ROWS_PER_W, IDX_ROWS_PER_W)],
                    idx_v)

    def chunk_body(t, carry):
        copies = []
        for j in range(CHUNK_STREAMS):
            copies.append(pltpu.async_copy(
                table_hbm.at[idx_v.at[t * CHUNK_STREAMS + j]],
                rows_v.at[pl.ds(j * STREAM_ROWS, STREAM_ROWS)],
                sem))
        for c in copies:
            c.wait()
        pltpu.sync_copy(rows_v, out_hbm.at[pl.ds(base + t * CHUNK_ROWS,
                                                 CHUNK_ROWS)])
        return carry

    lax.fori_loop(0, N_CHUNKS, chunk_body, 0)


def _sc_gather(table, idx2d):
    k = pl.kernel(
        _gather_body,
        out_type=jax.ShapeDtypeStruct((B, D), jnp.float32),
        mesh=plsc.VectorSubcoreMesh(core_axis_name="c", subcore_axis_name="s"),
        scratch_types=[
            pltpu.VMEM((IDX_ROWS_PER_W, STREAM_ROWS), jnp.int32),
            pltpu.VMEM((CHUNK_ROWS, D), jnp.float32),
            pltpu.SemaphoreType.DMA,
        ],
        compiler_params=pltpu.CompilerParams(use_tc_tiling_on_sc=False),
    )
    return k(table, idx2d)


BM = 4096


def _mlp_body(x_ref, w1t, w2t, w3t, b1, b2, b3, o_ref):
    h = jnp.dot(x_ref[...], w1t[...], precision=lax.Precision.HIGHEST,
                preferred_element_type=jnp.float32) + b1[...]
    h = jnp.dot(h, w2t[...], precision=lax.Precision.HIGHEST,
                preferred_element_type=jnp.float32) + b2[...]
    o_ref[...] = jnp.dot(h, w3t[...], precision=lax.Precision.HIGHEST,
                         preferred_element_type=jnp.float32) + b3[...]


def _tc_mlp(x, w1t, w2t, w3t, b1, b2, b3):
    wspec = pl.BlockSpec((D, D), lambda i: (0, 0))
    bspec = pl.BlockSpec((1, D), lambda i: (0, 0))
    return pl.pallas_call(
        _mlp_body,
        grid=(B // BM,),
        in_specs=[pl.BlockSpec((BM, D), lambda i: (i, 0)),
                  wspec, wspec, wspec, bspec, bspec, bspec],
        out_specs=pl.BlockSpec((BM, D), lambda i: (i, 0)),
        out_shape=jax.ShapeDtypeStruct((B, D), jnp.float32),
    )(x, w1t, w2t, w3t, b1, b2, b3)


def kernel(features_values, table, W1, b1, W2, b2, W3, b3):
    idx2d = features_values.astype(jnp.int32).reshape(B // STREAM_ROWS,
                                                      STREAM_ROWS)
    emb = _sc_gather(table, idx2d)
    return _tc_mlp(emb, W1.T, W2.T, W3.T,
                   b1.reshape(1, D), b2.reshape(1, D), b3.reshape(1, D))


# speedup vs baseline: 1.1949x; 1.1949x over previous
import jax
import jax.numpy as jnp
from jax import lax
from jax.experimental import pallas as pl
from jax.experimental.pallas import tpu as pltpu
from jax.experimental.pallas import tpu_sc as plsc

B = 327680
D = 64
V = 1000000
NUM_CORES = 2
NUM_SUBCORES = 16
NW = NUM_CORES * NUM_SUBCORES
ROWS_PER_W = B // NW
STREAM_ROWS = 128
CHUNK_STREAMS = 8
CHUNK_ROWS = STREAM_ROWS * CHUNK_STREAMS
N_CHUNKS = ROWS_PER_W // CHUNK_ROWS
IDX_ROWS_PER_W = ROWS_PER_W // STREAM_ROWS

BN = 4096
BM = 4096


def _transform_body(tt_ref, w1, w2, w3, b1, b2, b3, o_ref):
    p = jnp.dot(w3[...], jnp.dot(w2[...], w1[...],
                                 preferred_element_type=jnp.float32),
                preferred_element_type=jnp.float32)
    bc = lax.dot_general(b1[...], w2[...], (((1,), (1,)), ((), ())),
                         preferred_element_type=jnp.float32) + b2[...]
    bc = lax.dot_general(bc, w3[...], (((1,), (1,)), ((), ())),
                         preferred_element_type=jnp.float32) + b3[...]
    o_ref[...] = lax.dot_general(
        tt_ref[...], p, (((0,), (1,)), ((), ())),
        precision=lax.Precision.HIGHEST,
        preferred_element_type=jnp.float32) + bc


def _tc_transform(tableT, w1, w2, w3, b1, b2, b3):
    wspec = pl.BlockSpec((D, D), lambda i: (0, 0))
    bspec = pl.BlockSpec((1, D), lambda i: (0, 0))
    return pl.pallas_call(
        _transform_body,
        grid=(pl.cdiv(V, BN),),
        in_specs=[pl.BlockSpec((D, BN), lambda i: (0, i)),
                  wspec, wspec, wspec, bspec, bspec, bspec],
        out_specs=pl.BlockSpec((BN, D), lambda i: (i, 0)),
        out_shape=jax.ShapeDtypeStruct((V, D), jnp.float32),
    )(tableT, w1, w2, w3, b1, b2, b3)


def _gather_body(table_hbm, idx_hbm, out_hbm, idx_v, rows_v, sem):
    wid = lax.axis_index("s") * NUM_CORES + lax.axis_index("c")
    base = wid * ROWS_PER_W
    pltpu.sync_copy(idx_hbm.at[pl.ds(wid * IDX_ROWS_PER_W, IDX_ROWS_PER_W)],
                    idx_v)

    def chunk_body(t, carry):
        copies = []
        for j in range(CHUNK_STREAMS):
            copies.append(pltpu.async_copy(
                table_hbm.at[idx_v.at[t * CHUNK_STREAMS + j]],
                rows_v.at[pl.ds(j * STREAM_ROWS, STREAM_ROWS)],
                sem))
        for c in copies:
            c.wait()
        pltpu.sync_copy(rows_v, out_hbm.at[pl.ds(base + t * CHUNK_ROWS,
                                                 CHUNK_ROWS)])
        return carry

    lax.fori_loop(0, N_CHUNKS, chunk_body, 0)


def _sc_gather(table, idx2d):
    k = pl.kernel(
        _gather_body,
        out_type=jax.ShapeDtypeStruct((B, D), jnp.float32),
        mesh=plsc.VectorSubcoreMesh(core_axis_name="c", subcore_axis_name="s"),
        scratch_types=[
            pltpu.VMEM((IDX_ROWS_PER_W, STREAM_ROWS), jnp.int32),
            pltpu.VMEM((CHUNK_ROWS, D), jnp.float32),
            pltpu.SemaphoreType.DMA,
        ],
        compiler_params=pltpu.CompilerParams(use_tc_tiling_on_sc=False),
    )
    return k(table, idx2d)


def _transpose_body(x_ref, o_ref):
    eye = jnp.float32(1.0) * (lax.broadcasted_iota(jnp.int32, (D, D), 0) ==
                              lax.broadcasted_iota(jnp.int32, (D, D), 1))
    o_ref[...] = lax.dot_general(eye, x_ref[...], (((1,), (1,)), ((), ())),
                                 precision=lax.Precision.HIGHEST,
                                 preferred_element_type=jnp.float32)


def _tc_transpose(x):
    return pl.pallas_call(
        _transpose_body,
        grid=(B // BM,),
        in_specs=[pl.BlockSpec((BM, D), lambda i: (i, 0))],
        out_specs=pl.BlockSpec((D, BM), lambda i: (0, i)),
        out_shape=jax.ShapeDtypeStruct((D, B), jnp.float32),
    )(x)


def kernel(features_values, table, W1, b1, W2, b2, W3, b3):
    idx2d = features_values.astype(jnp.int32).reshape(B // STREAM_ROWS,
                                                      STREAM_ROWS)
    tableT = table.T
    twc = _tc_transform(tableT, W1, W2, W3,
                        b1.reshape(1, D), b2.reshape(1, D), b3.reshape(1, D))
    emb = _sc_gather(twc, idx2d)
    return _tc_transpose(emb).T


# speedup vs baseline: 1.5349x; 1.2846x over previous
import jax
import jax.numpy as jnp
from jax import lax
from jax.experimental import pallas as pl
from jax.experimental.pallas import tpu as pltpu
from jax.experimental.pallas import tpu_sc as plsc

B = 327680
D = 64
V = 1000000
NUM_CORES = 2
NUM_SUBCORES = 16
NW = NUM_CORES * NUM_SUBCORES
ROWS_PER_W = B // NW
STREAM_ROWS = 128
CHUNK_STREAMS = 8
CHUNK_ROWS = STREAM_ROWS * CHUNK_STREAMS
N_CHUNKS = ROWS_PER_W // CHUNK_ROWS
IDX_ROWS_PER_W = ROWS_PER_W // STREAM_ROWS

BN = 4096
BM = 4096


def _transform_body(tt_ref, w1, w2, w3, b1, b2, b3, o_ref):
    p = jnp.dot(w3[...], jnp.dot(w2[...], w1[...],
                                 preferred_element_type=jnp.float32),
                preferred_element_type=jnp.float32)
    bc = lax.dot_general(b1[...], w2[...], (((1,), (1,)), ((), ())),
                         preferred_element_type=jnp.float32) + b2[...]
    bc = lax.dot_general(bc, w3[...], (((1,), (1,)), ((), ())),
                         preferred_element_type=jnp.float32) + b3[...]
    o_ref[...] = lax.dot_general(
        tt_ref[...], p, (((0,), (1,)), ((), ())),
        preferred_element_type=jnp.float32) + bc


def _tc_transform(tableT, w1, w2, w3, b1, b2, b3):
    wspec = pl.BlockSpec((D, D), lambda i: (0, 0))
    bspec = pl.BlockSpec((1, D), lambda i: (0, 0))
    return pl.pallas_call(
        _transform_body,
        grid=(pl.cdiv(V, BN),),
        in_specs=[pl.BlockSpec((D, BN), lambda i: (0, i)),
                  wspec, wspec, wspec, bspec, bspec, bspec],
        out_specs=pl.BlockSpec((BN, D), lambda i: (i, 0)),
        out_shape=jax.ShapeDtypeStruct((V, D), jnp.float32),
    )(tableT, w1, w2, w3, b1, b2, b3)


def _gather_body(table_hbm, idx_hbm, out_hbm, idx_v, rows_v, sem):
    wid = lax.axis_index("s") * NUM_CORES + lax.axis_index("c")
    base = wid * ROWS_PER_W
    pltpu.sync_copy(idx_hbm.at[pl.ds(wid * IDX_ROWS_PER_W, IDX_ROWS_PER_W)],
                    idx_v)

    def chunk_body(t, carry):
        copies = []
        for j in range(CHUNK_STREAMS):
            copies.append(pltpu.async_copy(
                table_hbm.at[idx_v.at[t * CHUNK_STREAMS + j]],
                rows_v.at[pl.ds(j * STREAM_ROWS, STREAM_ROWS)],
                sem))
        for c in copies:
            c.wait()
        pltpu.sync_copy(rows_v, out_hbm.at[pl.ds(base + t * CHUNK_ROWS,
                                                 CHUNK_ROWS)])
        return carry

    lax.fori_loop(0, N_CHUNKS, chunk_body, 0)


def _sc_gather(table, idx2d):
    k = pl.kernel(
        _gather_body,
        out_type=jax.ShapeDtypeStruct((B, D), jnp.float32),
        mesh=plsc.VectorSubcoreMesh(core_axis_name="c", subcore_axis_name="s"),
        scratch_types=[
            pltpu.VMEM((IDX_ROWS_PER_W, STREAM_ROWS), jnp.int32),
            pltpu.VMEM((CHUNK_ROWS, D), jnp.float32),
            pltpu.SemaphoreType.DMA,
        ],
        compiler_params=pltpu.CompilerParams(use_tc_tiling_on_sc=False),
    )
    return k(table, idx2d)


def _transpose_body(x_ref, o_ref):
    eye = jnp.float32(1.0) * (lax.broadcasted_iota(jnp.int32, (D, D), 0) ==
                              lax.broadcasted_iota(jnp.int32, (D, D), 1))
    o_ref[...] = lax.dot_general(eye, x_ref[...], (((1,), (1,)), ((), ())),
                                 precision=lax.Precision.HIGHEST,
                                 preferred_element_type=jnp.float32)


def _tc_transpose(x):
    return pl.pallas_call(
        _transpose_body,
        grid=(B // BM,),
        in_specs=[pl.BlockSpec((BM, D), lambda i: (i, 0))],
        out_specs=pl.BlockSpec((D, BM), lambda i: (0, i)),
        out_shape=jax.ShapeDtypeStruct((D, B), jnp.float32),
    )(x)


def kernel(features_values, table, W1, b1, W2, b2, W3, b3):
    idx2d = features_values.astype(jnp.int32).reshape(B // STREAM_ROWS,
                                                      STREAM_ROWS)
    tableT = table.T
    twc = _tc_transform(tableT, W1, W2, W3,
                        b1.reshape(1, D), b2.reshape(1, D), b3.reshape(1, D))
    emb = _sc_gather(twc, idx2d)
    return emb


# speedup vs baseline: 2.5441x; 1.6575x over previous
import jax
import jax.numpy as jnp
from jax import lax
from jax.experimental import pallas as pl
from jax.experimental.pallas import tpu as pltpu
from jax.experimental.pallas import tpu_sc as plsc

B = 327680
D = 64
V = 1000000
BS = 2048
GRID_A = 245
VPAD = GRID_A * 2 * BS
HALF = B // 2

NUM_CORES = 2
NUM_SUBCORES = 16
NW = NUM_CORES * NUM_SUBCORES
ROWS_PER_W = B // NW
STREAM_ROWS = 128
CHUNK_STREAMS = 8
CHUNK_ROWS = STREAM_ROWS * CHUNK_STREAMS
N_CHUNKS = ROWS_PER_W // CHUNK_ROWS
IDX_ROWS_PER_W = ROWS_PER_W // STREAM_ROWS
REMAP_VECS = ROWS_PER_W // 16

BQ = 2048


def _transform_body(t0_ref, t1_ref, w1, w2, w3, b1, b2, b3, o_ref):
    p = jnp.dot(w3[...], jnp.dot(w2[...], w1[...],
                                 preferred_element_type=jnp.float32),
                preferred_element_type=jnp.float32)
    bc = lax.dot_general(b1[...], w2[...], (((1,), (1,)), ((), ())),
                         preferred_element_type=jnp.float32) + b2[...]
    bc = lax.dot_general(bc, w3[...], (((1,), (1,)), ((), ())),
                         preferred_element_type=jnp.float32) + b3[...]
    o_ref[:, :D] = lax.dot_general(
        t0_ref[...], p, (((0,), (1,)), ((), ())),
        preferred_element_type=jnp.float32) + bc
    o_ref[:, D:] = lax.dot_general(
        t1_ref[...], p, (((0,), (1,)), ((), ())),
        preferred_element_type=jnp.float32) + bc


def _tc_transform(tableT, w1, w2, w3, b1, b2, b3):
    wspec = pl.BlockSpec((D, D), lambda c: (0, 0))
    bspec = pl.BlockSpec((1, D), lambda c: (0, 0))
    return pl.pallas_call(
        _transform_body,
        grid=(GRID_A,),
        in_specs=[pl.BlockSpec((D, BS), lambda c: (0, 2 * c)),
                  pl.BlockSpec((D, BS),
                               lambda c: (0, jnp.minimum(2 * c + 1,
                                                         2 * GRID_A - 2))),
                  wspec, wspec, wspec, bspec, bspec, bspec],
        out_specs=pl.BlockSpec((BS, 2 * D), lambda c: (c, 0)),
        out_shape=jax.ShapeDtypeStruct((GRID_A * BS, 2 * D), jnp.float32),
    )(tableT, tableT, w1, w2, w3, b1, b2, b3)


def _gather_body(table_hbm, idx_hbm, out_hbm, idx_v, rows_v, sem):
    wid = lax.axis_index("s") * NUM_CORES + lax.axis_index("c")
    base = wid * ROWS_PER_W
    pltpu.sync_copy(idx_hbm.at[pl.ds(wid * IDX_ROWS_PER_W, IDX_ROWS_PER_W)],
                    idx_v)

    def remap_body(t, carry):
        row = t // 8
        g = (t % 8) * 16
        i = idx_v[row, pl.ds(g, 16)]
        r = ((i >> 12) << 12) + ((i & 2047) << 1) + ((i >> 11) & 1)
        idx_v[row, pl.ds(g, 16)] = r
        return carry

    lax.fori_loop(0, REMAP_VECS, remap_body, 0)

    def chunk_body(t, carry):
        copies = []
        for j in range(CHUNK_STREAMS):
            copies.append(pltpu.async_copy(
                table_hbm.at[idx_v.at[t * CHUNK_STREAMS + j]],
                rows_v.at[pl.ds(j * STREAM_ROWS, STREAM_ROWS)],
                sem))
        for c in copies:
            c.wait()
        pltpu.sync_copy(rows_v,
                        out_hbm.at[pl.ds(base + t * CHUNK_ROWS, CHUNK_ROWS)])
        return carry

    lax.fori_loop(0, N_CHUNKS, chunk_body, 0)


def _sc_gather(table_flat, idx2d):
    k = pl.kernel(
        _gather_body,
        out_type=jax.ShapeDtypeStruct((B, D), jnp.float32),
        mesh=plsc.VectorSubcoreMesh(core_axis_name="c", subcore_axis_name="s"),
        scratch_types=[
            pltpu.VMEM((IDX_ROWS_PER_W, STREAM_ROWS), jnp.int32),
            pltpu.VMEM((CHUNK_ROWS, D), jnp.float32),
            pltpu.SemaphoreType.DMA,
        ],
        compiler_params=pltpu.CompilerParams(use_tc_tiling_on_sc=False),
    )
    return k(table_flat, idx2d)


def _transpose_body(x_ref, o_ref):
    j = pl.program_id(1)
    xh = jnp.where(j == 0, x_ref[:, :D], x_ref[:, D:])
    eye = jnp.float32(1.0) * (lax.broadcasted_iota(jnp.int32, (D, D), 0) ==
                              lax.broadcasted_iota(jnp.int32, (D, D), 1))
    o_ref[...] = lax.dot_general(eye, xh, (((1,), (1,)), ((), ())),
                                 preferred_element_type=jnp.float32)


def _tc_transpose(xp):
    return pl.pallas_call(
        _transpose_body,
        grid=(HALF // BQ, 2),
        in_specs=[pl.BlockSpec((BQ, 2 * D), lambda c, j: (c, 0))],
        out_specs=pl.BlockSpec((D, BQ), lambda c, j: (0, j * (HALF // BQ) + c)),
        out_shape=jax.ShapeDtypeStruct((D, B), jnp.float32),
    )(xp)


def kernel(features_values, table, W1, b1, W2, b2, W3, b3):
    idx2d = features_values.astype(jnp.int32).reshape(B // STREAM_ROWS,
                                                      STREAM_ROWS)
    tableT = table.T
    twc = _tc_transform(tableT, W1, W2, W3,
                        b1.reshape(1, D), b2.reshape(1, D), b3.reshape(1, D))
    twc_flat = twc.reshape(VPAD, D)
    emb = _sc_gather(twc_flat, idx2d)
    return emb


# speedup vs baseline: 2.6827x; 1.0545x over previous
import jax
import jax.numpy as jnp
from jax import lax
from jax.experimental import pallas as pl
from jax.experimental.pallas import tpu as pltpu
from jax.experimental.pallas import tpu_sc as plsc

B = 327680
D = 64
V = 1000000
BS = 2048
GRID_A = 245
VPAD = GRID_A * 2 * BS
HALF = B // 2

NUM_CORES = 2
NUM_SUBCORES = 16
NW = NUM_CORES * NUM_SUBCORES
ROWS_PER_W = B // NW
STREAM_ROWS = 128
CHUNK_STREAMS = 8
CHUNK_ROWS = STREAM_ROWS * CHUNK_STREAMS
N_CHUNKS = ROWS_PER_W // CHUNK_ROWS
IDX_ROWS_PER_W = ROWS_PER_W // STREAM_ROWS
REMAP_VECS = ROWS_PER_W // 16

BQ = 2048


def _transform_body(t0_ref, t1_ref, w1, w2, w3, b1, b2, b3, o_ref):
    p = jnp.dot(w3[...], jnp.dot(w2[...], w1[...],
                                 preferred_element_type=jnp.float32),
                preferred_element_type=jnp.float32)
    bc = lax.dot_general(b1[...], w2[...], (((1,), (1,)), ((), ())),
                         preferred_element_type=jnp.float32) + b2[...]
    bc = lax.dot_general(bc, w3[...], (((1,), (1,)), ((), ())),
                         preferred_element_type=jnp.float32) + b3[...]
    o_ref[:, :D] = lax.dot_general(
        t0_ref[...], p, (((0,), (1,)), ((), ())),
        preferred_element_type=jnp.float32) + bc
    o_ref[:, D:] = lax.dot_general(
        t1_ref[...], p, (((0,), (1,)), ((), ())),
        preferred_element_type=jnp.float32) + bc


def _tc_transform(tableT, w1, w2, w3, b1, b2, b3):
    wspec = pl.BlockSpec((D, D), lambda c: (0, 0))
    bspec = pl.BlockSpec((1, D), lambda c: (0, 0))
    return pl.pallas_call(
        _transform_body,
        grid=(GRID_A,),
        in_specs=[pl.BlockSpec((D, BS), lambda c: (0, 2 * c)),
                  pl.BlockSpec((D, BS),
                               lambda c: (0, jnp.minimum(2 * c + 1,
                                                         2 * GRID_A - 2))),
                  wspec, wspec, wspec, bspec, bspec, bspec],
        out_specs=pl.BlockSpec((BS, 2 * D), lambda c: (c, 0)),
        out_shape=jax.ShapeDtypeStruct((GRID_A * BS, 2 * D), jnp.float32),
    )(tableT, tableT, w1, w2, w3, b1, b2, b3)


def _gather_body(table_hbm, idx_hbm, out_hbm, idx_v, rows_v, sem):
    wid = lax.axis_index("s") * NUM_CORES + lax.axis_index("c")
    half = wid // NUM_SUBCORES
    qbase = (wid % NUM_SUBCORES) * ROWS_PER_W
    pltpu.sync_copy(idx_hbm.at[pl.ds(wid * IDX_ROWS_PER_W, IDX_ROWS_PER_W)],
                    idx_v)

    def remap_body(t, carry):
        row = t // 8
        g = (t % 8) * 16
        i = idx_v[row, pl.ds(g, 16)]
        r = ((i >> 12) << 12) + ((i & 2047) << 1) + ((i >> 11) & 1)
        idx_v[row, pl.ds(g, 16)] = r
        return carry

    lax.fori_loop(0, REMAP_VECS, remap_body, 0)

    def chunk_body(t, carry):
        copies = []
        for j in range(CHUNK_STREAMS):
            copies.append(pltpu.async_copy(
                table_hbm.at[idx_v.at[t * CHUNK_STREAMS + j]],
                rows_v.at[pl.ds(j * STREAM_ROWS, STREAM_ROWS)],
                sem))
        for c in copies:
            c.wait()
        pltpu.sync_copy(rows_v,
                        out_hbm.at[pl.ds(qbase + t * CHUNK_ROWS, CHUNK_ROWS),
                                   pl.ds(half * D, D)])
        return carry

    lax.fori_loop(0, N_CHUNKS, chunk_body, 0)


def _sc_gather(table_flat, idx2d):
    k = pl.kernel(
        _gather_body,
        out_type=jax.ShapeDtypeStruct((HALF, 2 * D), jnp.float32),
        mesh=plsc.VectorSubcoreMesh(core_axis_name="c", subcore_axis_name="s"),
        scratch_types=[
            pltpu.VMEM((IDX_ROWS_PER_W, STREAM_ROWS), jnp.int32),
            pltpu.VMEM((CHUNK_ROWS, D), jnp.float32),
            pltpu.SemaphoreType.DMA,
        ],
        compiler_params=pltpu.CompilerParams(use_tc_tiling_on_sc=False),
    )
    return k(table_flat, idx2d)


def _transpose_body(x_ref, o_ref):
    j = pl.program_id(1)
    xh = jnp.where(j == 0, x_ref[:, :D], x_ref[:, D:])
    eye = jnp.float32(1.0) * (lax.broadcasted_iota(jnp.int32, (D, D), 0) ==
                              lax.broadcasted_iota(jnp.int32, (D, D), 1))
    o_ref[...] = lax.dot_general(eye, xh, (((1,), (1,)), ((), ())),
                                 preferred_element_type=jnp.float32)


def _tc_transpose(xp):
    return pl.pallas_call(
        _transpose_body,
        grid=(HALF // BQ, 2),
        in_specs=[pl.BlockSpec((BQ, 2 * D), lambda c, j: (c, 0))],
        out_specs=pl.BlockSpec((D, BQ), lambda c, j: (0, j * (HALF // BQ) + c)),
        out_shape=jax.ShapeDtypeStruct((D, B), jnp.float32),
    )(xp)


def kernel(features_values, table, W1, b1, W2, b2, W3, b3):
    idx2d = features_values.astype(jnp.int32).reshape(B // STREAM_ROWS,
                                                      STREAM_ROWS)
    tableT = table.T
    twc = _tc_transform(tableT, W1, W2, W3,
                        b1.reshape(1, D), b2.reshape(1, D), b3.reshape(1, D))
    twc_flat = twc.reshape(VPAD, D)
    emb = _sc_gather(twc_flat, idx2d)
    return _tc_transpose(emb).T


# speedup vs baseline: 3.1868x; 1.1879x over previous
import jax
import jax.numpy as jnp
from jax import lax
from jax.experimental import pallas as pl
from jax.experimental.pallas import tpu as pltpu
from jax.experimental.pallas import tpu_sc as plsc

B = 327680
D = 64
V = 1000000
BS = 2048
GRID_A = 245
VPAD = GRID_A * 2 * BS
HALF = B // 2

NUM_CORES = 2
NUM_SUBCORES = 16
NW = NUM_CORES * NUM_SUBCORES
ROWS_PER_W = B // NW
STREAM_ROWS = 128
CHUNK_STREAMS = 8
CHUNK_ROWS = STREAM_ROWS * CHUNK_STREAMS
N_CHUNKS = ROWS_PER_W // CHUNK_ROWS
IDX_ROWS_PER_W = ROWS_PER_W // STREAM_ROWS
REMAP_VECS = ROWS_PER_W // 16

BQ = 2048


def _transform_body(t0_ref, t1_ref, w1, w2, w3, b1, b2, b3, o_ref):
    p = jnp.dot(w3[...], jnp.dot(w2[...], w1[...],
                                 preferred_element_type=jnp.float32),
                preferred_element_type=jnp.float32)
    bc = lax.dot_general(b1[...], w2[...], (((1,), (1,)), ((), ())),
                         preferred_element_type=jnp.float32) + b2[...]
    bc = lax.dot_general(bc, w3[...], (((1,), (1,)), ((), ())),
                         preferred_element_type=jnp.float32) + b3[...]
    pb = p.astype(jnp.bfloat16)
    o_ref[:, :D] = lax.dot_general(
        t0_ref[...].astype(jnp.bfloat16), pb, (((0,), (1,)), ((), ())),
        preferred_element_type=jnp.float32) + bc
    o_ref[:, D:] = lax.dot_general(
        t1_ref[...].astype(jnp.bfloat16), pb, (((0,), (1,)), ((), ())),
        preferred_element_type=jnp.float32) + bc


def _tc_transform(tableT, w1, w2, w3, b1, b2, b3):
    wspec = pl.BlockSpec((D, D), lambda c: (0, 0))
    bspec = pl.BlockSpec((1, D), lambda c: (0, 0))
    return pl.pallas_call(
        _transform_body,
        grid=(GRID_A,),
        in_specs=[pl.BlockSpec((D, BS), lambda c: (0, 2 * c)),
                  pl.BlockSpec((D, BS),
                               lambda c: (0, jnp.minimum(2 * c + 1,
                                                         2 * GRID_A - 2))),
                  wspec, wspec, wspec, bspec, bspec, bspec],
        out_specs=pl.BlockSpec((BS, 2 * D), lambda c: (c, 0)),
        out_shape=jax.ShapeDtypeStruct((GRID_A * BS, 2 * D), jnp.float32),
    )(tableT, tableT, w1, w2, w3, b1, b2, b3)


def _gather_body(table_hbm, idx_hbm, out_hbm, idx_v, rows_v, sem):
    wid = lax.axis_index("s") * NUM_CORES + lax.axis_index("c")
    nbase = wid * ROWS_PER_W
    pltpu.sync_copy(idx_hbm.at[pl.ds(wid * IDX_ROWS_PER_W, IDX_ROWS_PER_W)],
                    idx_v)

    def remap_body(t, carry):
        row = t // 8
        g = (t % 8) * 16
        i = idx_v[row, pl.ds(g, 16)]
        r = ((i >> 12) << 12) + ((i & 2047) << 1) + ((i >> 11) & 1)
        idx_v[row, pl.ds(g, 16)] = r
        return carry

    lax.fori_loop(0, REMAP_VECS, remap_body, 0)

    def chunk_body(t, carry):
        copies = []
        for j in range(CHUNK_STREAMS):
            copies.append(pltpu.async_copy(
                table_hbm.at[idx_v.at[t * CHUNK_STREAMS + j]],
                rows_v.at[pl.ds(j * STREAM_ROWS, STREAM_ROWS)],
                sem))
        for c in copies:
            c.wait()
        n0 = nbase + t * CHUNK_ROWS
        q0 = ((n0 >> 12) << 11) + (n0 & 2047)
        k = (n0 >> 11) & 1
        pltpu.sync_copy(rows_v,
                        out_hbm.at[pl.ds(q0, CHUNK_ROWS),
                                   pl.ds(k * D, D)])
        return carry

    lax.fori_loop(0, N_CHUNKS, chunk_body, 0)


def _sc_gather(table_flat, idx2d):
    k = pl.kernel(
        _gather_body,
        out_type=jax.ShapeDtypeStruct((HALF, 2 * D), jnp.float32),
        mesh=plsc.VectorSubcoreMesh(core_axis_name="c", subcore_axis_name="s"),
        scratch_types=[
            pltpu.VMEM((IDX_ROWS_PER_W, STREAM_ROWS), jnp.int32),
            pltpu.VMEM((CHUNK_ROWS, D), jnp.float32),
            pltpu.SemaphoreType.DMA,
        ],
        compiler_params=pltpu.CompilerParams(use_tc_tiling_on_sc=False),
    )
    return k(table_flat, idx2d)


def _transpose_body(x_ref, o_ref):
    eye = jnp.float32(1.0) * (lax.broadcasted_iota(jnp.int32, (D, D), 0) ==
                              lax.broadcasted_iota(jnp.int32, (D, D), 1))
    o_ref[:, :BQ] = lax.dot_general(eye, x_ref[:, :D], (((1,), (1,)), ((), ())),
                                    preferred_element_type=jnp.float32)
    o_ref[:, BQ:] = lax.dot_general(eye, x_ref[:, D:], (((1,), (1,)), ((), ())),
                                    preferred_element_type=jnp.float32)


def _tc_transpose(xp):
    return pl.pallas_call(
        _transpose_body,
        grid=(HALF // BQ,),
        in_specs=[pl.BlockSpec((BQ, 2 * D), lambda c: (c, 0))],
        out_specs=pl.BlockSpec((D, 2 * BQ), lambda c: (0, c)),
        out_shape=jax.ShapeDtypeStruct((D, B), jnp.float32),
    )(xp)


def kernel(features_values, table, W1, b1, W2, b2, W3, b3):
    idx2d = features_values.astype(jnp.int32).reshape(B // STREAM_ROWS,
                                                      STREAM_ROWS)
    tableT = table.T
    twc = _tc_transform(tableT, W1, W2, W3,
                        b1.reshape(1, D), b2.reshape(1, D), b3.reshape(1, D))
    twc_flat = twc.reshape(VPAD, D)
    emb = _sc_gather(twc_flat, idx2d)
    return _tc_transpose(emb).T


# speedup vs baseline: 4.0657x; 1.2758x over previous
import jax
import jax.numpy as jnp
from jax import lax
from jax.experimental import pallas as pl
from jax.experimental.pallas import tpu as pltpu
from jax.experimental.pallas import tpu_sc as plsc

B = 327680
D = 64
V = 1000000
BS = 4096
GRID_A = 123
VPAD = GRID_A * 2 * BS
HALF = B // 2

NUM_CORES = 2
NUM_SUBCORES = 16
NW = NUM_CORES * NUM_SUBCORES
ROWS_PER_W = B // NW
STREAM_ROWS = 128
CHUNK_STREAMS = 8
CHUNK_ROWS = STREAM_ROWS * CHUNK_STREAMS
N_CHUNKS = ROWS_PER_W // CHUNK_ROWS
IDX_ROWS_PER_W = ROWS_PER_W // STREAM_ROWS
REMAP_VECS = ROWS_PER_W // 16

BQ = 4096


def _transform_body(t0_ref, t1_ref, w1, w2, w3, b1, b2, b3, o_ref):
    p = jnp.dot(w3[...], jnp.dot(w2[...], w1[...],
                                 preferred_element_type=jnp.float32),
                preferred_element_type=jnp.float32)
    bc = lax.dot_general(b1[...], w2[...], (((1,), (1,)), ((), ())),
                         preferred_element_type=jnp.float32) + b2[...]
    bc = lax.dot_general(bc, w3[...], (((1,), (1,)), ((), ())),
                         preferred_element_type=jnp.float32) + b3[...]
    pb = p.astype(jnp.bfloat16)
    o_ref[:, :D] = lax.dot_general(
        t0_ref[...].astype(jnp.bfloat16), pb, (((0,), (1,)), ((), ())),
        preferred_element_type=jnp.float32) + bc
    o_ref[:, D:] = lax.dot_general(
        t1_ref[...].astype(jnp.bfloat16), pb, (((0,), (1,)), ((), ())),
        preferred_element_type=jnp.float32) + bc


def _tc_transform(tableT, w1, w2, w3, b1, b2, b3):
    wspec = pl.BlockSpec((D, D), lambda c: (0, 0))
    bspec = pl.BlockSpec((1, D), lambda c: (0, 0))
    return pl.pallas_call(
        _transform_body,
        grid=(GRID_A,),
        in_specs=[pl.BlockSpec((D, BS), lambda c: (0, 2 * c)),
                  pl.BlockSpec((D, BS),
                               lambda c: (0, jnp.minimum(2 * c + 1,
                                                         2 * GRID_A - 2))),
                  wspec, wspec, wspec, bspec, bspec, bspec],
        out_specs=pl.BlockSpec((BS, 2 * D), lambda c: (c, 0)),
        out_shape=jax.ShapeDtypeStruct((GRID_A * BS, 2 * D), jnp.float32),
    )(tableT, tableT, w1, w2, w3, b1, b2, b3)


def _gather_body(table_hbm, idx_hbm, out_hbm, idx_v, rows_v, sem):
    wid = lax.axis_index("s") * NUM_CORES + lax.axis_index("c")
    nbase = wid * ROWS_PER_W
    pltpu.sync_copy(idx_hbm.at[pl.ds(wid * IDX_ROWS_PER_W, IDX_ROWS_PER_W)],
                    idx_v)

    def remap_body(t, carry):
        row = t // 8
        g = (t % 8) * 16
        i = idx_v[row, pl.ds(g, 16)]
        r = ((i >> 13) << 13) + ((i & 4095) << 1) + ((i >> 12) & 1)
        idx_v[row, pl.ds(g, 16)] = r
        return carry

    lax.fori_loop(0, REMAP_VECS, remap_body, 0)

    def chunk_body(t, carry):
        copies = []
        for j in range(CHUNK_STREAMS):
            copies.append(pltpu.async_copy(
                table_hbm.at[idx_v.at[t * CHUNK_STREAMS + j]],
                rows_v.at[pl.ds(j * STREAM_ROWS, STREAM_ROWS)],
                sem))
        for c in copies:
            c.wait()
        n0 = nbase + t * CHUNK_ROWS
        q0 = ((n0 >> 12) << 11) + (n0 & 2047)
        k = (n0 >> 11) & 1
        pltpu.sync_copy(rows_v,
                        out_hbm.at[pl.ds(q0, CHUNK_ROWS),
                                   pl.ds(k * D, D)])
        return carry

    lax.fori_loop(0, N_CHUNKS, chunk_body, 0)


def _sc_gather(table_flat, idx2d):
    k = pl.kernel(
        _gather_body,
        out_type=jax.ShapeDtypeStruct((HALF, 2 * D), jnp.float32),
        mesh=plsc.VectorSubcoreMesh(core_axis_name="c", subcore_axis_name="s"),
        scratch_types=[
            pltpu.VMEM((IDX_ROWS_PER_W, STREAM_ROWS), jnp.int32),
            pltpu.VMEM((CHUNK_ROWS, D), jnp.float32),
            pltpu.SemaphoreType.DMA,
        ],
        compiler_params=pltpu.CompilerParams(use_tc_tiling_on_sc=False),
    )
    return k(table_flat, idx2d)


def _transpose_body(x_ref, o_ref):
    eye = jnp.float32(1.0) * (lax.broadcasted_iota(jnp.int32, (D, D), 0) ==
                              lax.broadcasted_iota(jnp.int32, (D, D), 1))
    for s in range(BQ // 2048):
        for k in range(2):
            o_ref[:, (2 * s + k) * 2048:(2 * s + k + 1) * 2048] = (
                lax.dot_general(eye,
                                x_ref[s * 2048:(s + 1) * 2048,
                                      k * D:(k + 1) * D],
                                (((1,), (1,)), ((), ())),
                                preferred_element_type=jnp.float32))


def _tc_transpose(xp):
    return pl.pallas_call(
        _transpose_body,
        grid=(HALF // BQ,),
        in_specs=[pl.BlockSpec((BQ, 2 * D), lambda c: (c, 0))],
        out_specs=pl.BlockSpec((D, 2 * BQ), lambda c: (0, c)),
        out_shape=jax.ShapeDtypeStruct((D, B), jnp.float32),
    )(xp)


def kernel(features_values, table, W1, b1, W2, b2, W3, b3):
    idx2d = features_values.astype(jnp.int32).reshape(B // STREAM_ROWS,
                                                      STREAM_ROWS)
    tableT = table.T
    twc = _tc_transform(tableT, W1, W2, W3,
                        b1.reshape(1, D), b2.reshape(1, D), b3.reshape(1, D))
    twc_flat = twc.reshape(VPAD, D)
    emb = _sc_gather(twc_flat, idx2d)
    return _tc_transpose(emb).T


# speedup vs baseline: 4.6482x; 1.1433x over previous
import jax
import jax.numpy as jnp
from jax import lax
from jax.experimental import pallas as pl
from jax.experimental.pallas import tpu as pltpu
from jax.experimental.pallas import tpu_sc as plsc

B = 327680
D = 64
V = 1000000
BS = 8192
GRID_A = 62
VPAD = GRID_A * 2 * BS
HALF = B // 2

NUM_CORES = 2
NUM_SUBCORES = 16
NW = NUM_CORES * NUM_SUBCORES
ROWS_PER_W = B // NW
STREAM_ROWS = 128
CHUNK_STREAMS = 8
CHUNK_ROWS = STREAM_ROWS * CHUNK_STREAMS
N_CHUNKS = ROWS_PER_W // CHUNK_ROWS
IDX_ROWS_PER_W = ROWS_PER_W // STREAM_ROWS
REMAP_VECS = ROWS_PER_W // 16

BQ = 8192


def _transform_body(t0_ref, t1_ref, w1, w2, w3, b1, b2, b3, o_ref):
    p = jnp.dot(w3[...], jnp.dot(w2[...], w1[...],
                                 preferred_element_type=jnp.float32),
                preferred_element_type=jnp.float32)
    bc = lax.dot_general(b1[...], w2[...], (((1,), (1,)), ((), ())),
                         preferred_element_type=jnp.float32) + b2[...]
    bc = lax.dot_general(bc, w3[...], (((1,), (1,)), ((), ())),
                         preferred_element_type=jnp.float32) + b3[...]
    pb = p.astype(jnp.bfloat16)
    o_ref[:, :D] = lax.dot_general(
        t0_ref[...].astype(jnp.bfloat16), pb, (((0,), (1,)), ((), ())),
        preferred_element_type=jnp.float32) + bc
    o_ref[:, D:] = lax.dot_general(
        t1_ref[...].astype(jnp.bfloat16), pb, (((0,), (1,)), ((), ())),
        preferred_element_type=jnp.float32) + bc


def _tc_transform(tableT, w1, w2, w3, b1, b2, b3):
    wspec = pl.BlockSpec((D, D), lambda c: (0, 0))
    bspec = pl.BlockSpec((1, D), lambda c: (0, 0))
    return pl.pallas_call(
        _transform_body,
        grid=(GRID_A,),
        in_specs=[pl.BlockSpec((D, BS), lambda c: (0, 2 * c)),
                  pl.BlockSpec((D, BS),
                               lambda c: (0, jnp.minimum(2 * c + 1,
                                                         2 * GRID_A - 2))),
                  wspec, wspec, wspec, bspec, bspec, bspec],
        out_specs=pl.BlockSpec((BS, 2 * D), lambda c: (c, 0)),
        out_shape=jax.ShapeDtypeStruct((GRID_A * BS, 2 * D), jnp.float32),
    )(tableT, tableT, w1, w2, w3, b1, b2, b3)


def _gather_body(table_hbm, idx_hbm, out_hbm, idx_v, rows_v, sem):
    wid = lax.axis_index("s") * NUM_CORES + lax.axis_index("c")
    nbase = wid * ROWS_PER_W
    pltpu.sync_copy(idx_hbm.at[pl.ds(wid * IDX_ROWS_PER_W, IDX_ROWS_PER_W)],
                    idx_v)

    def remap_body(t, carry):
        row = t // 8
        g = (t % 8) * 16
        i = idx_v[row, pl.ds(g, 16)]
        r = ((i >> 14) << 14) + ((i & 8191) << 1) + ((i >> 13) & 1)
        idx_v[row, pl.ds(g, 16)] = r
        return carry

    lax.fori_loop(0, REMAP_VECS, remap_body, 0)

    def chunk_body(t, carry):
        copies = []
        for j in range(CHUNK_STREAMS):
            copies.append(pltpu.async_copy(
                table_hbm.at[idx_v.at[t * CHUNK_STREAMS + j]],
                rows_v.at[pl.ds(j * STREAM_ROWS, STREAM_ROWS)],
                sem))
        for c in copies:
            c.wait()
        n0 = nbase + t * CHUNK_ROWS
        q0 = ((n0 >> 12) << 11) + (n0 & 2047)
        k = (n0 >> 11) & 1
        pltpu.sync_copy(rows_v,
                        out_hbm.at[pl.ds(q0, CHUNK_ROWS),
                                   pl.ds(k * D, D)])
        return carry

    lax.fori_loop(0, N_CHUNKS, chunk_body, 0)


def _sc_gather(table_flat, idx2d):
    k = pl.kernel(
        _gather_body,
        out_type=jax.ShapeDtypeStruct((HALF, 2 * D), jnp.float32),
        mesh=plsc.VectorSubcoreMesh(core_axis_name="c", subcore_axis_name="s"),
        scratch_types=[
            pltpu.VMEM((IDX_ROWS_PER_W, STREAM_ROWS), jnp.int32),
            pltpu.VMEM((CHUNK_ROWS, D), jnp.float32),
            pltpu.SemaphoreType.DMA,
        ],
        compiler_params=pltpu.CompilerParams(use_tc_tiling_on_sc=False),
    )
    return k(table_flat, idx2d)


def _transpose_body(x_ref, o_ref):
    eye = jnp.float32(1.0) * (lax.broadcasted_iota(jnp.int32, (D, D), 0) ==
                              lax.broadcasted_iota(jnp.int32, (D, D), 1))
    for s in range(BQ // 2048):
        for k in range(2):
            o_ref[:, (2 * s + k) * 2048:(2 * s + k + 1) * 2048] = (
                lax.dot_general(eye,
                                x_ref[s * 2048:(s + 1) * 2048,
                                      k * D:(k + 1) * D],
                                (((1,), (1,)), ((), ())),
                                preferred_element_type=jnp.float32))


def _tc_transpose(xp):
    return pl.pallas_call(
        _transpose_body,
        grid=(HALF // BQ,),
        in_specs=[pl.BlockSpec((BQ, 2 * D), lambda c: (c, 0))],
        out_specs=pl.BlockSpec((D, 2 * BQ), lambda c: (0, c)),
        out_shape=jax.ShapeDtypeStruct((D, B), jnp.float32),
    )(xp)


def kernel(features_values, table, W1, b1, W2, b2, W3, b3):
    idx2d = features_values.astype(jnp.int32).reshape(B // STREAM_ROWS,
                                                      STREAM_ROWS)
    tableT = table.T
    twc = _tc_transform(tableT, W1, W2, W3,
                        b1.reshape(1, D), b2.reshape(1, D), b3.reshape(1, D))
    twc_flat = twc.reshape(VPAD, D)
    emb = _sc_gather(twc_flat, idx2d)
    return _tc_transpose(emb).T


# speedup vs baseline: 4.9483x; 1.0646x over previous
import jax
import jax.numpy as jnp
from jax import lax
from jax.experimental import pallas as pl
from jax.experimental.pallas import tpu as pltpu
from jax.experimental.pallas import tpu_sc as plsc

B = 327680
D = 64
V = 1000000
BS = 16384
GRID_A = 31
VPAD = GRID_A * 2 * BS
HALF = B // 2

NUM_CORES = 2
NUM_SUBCORES = 16
NW = NUM_CORES * NUM_SUBCORES
ROWS_PER_W = B // NW
STREAM_ROWS = 128
CHUNK_STREAMS = 8
CHUNK_ROWS = STREAM_ROWS * CHUNK_STREAMS
N_CHUNKS = ROWS_PER_W // CHUNK_ROWS
IDX_ROWS_PER_W = ROWS_PER_W // STREAM_ROWS
REMAP_VECS = ROWS_PER_W // 16

BQ = 16384


def _transform_body(t0_ref, t1_ref, w1, w2, w3, b1, b2, b3, o_ref):
    p = jnp.dot(w3[...], jnp.dot(w2[...], w1[...],
                                 preferred_element_type=jnp.float32),
                preferred_element_type=jnp.float32)
    bc = lax.dot_general(b1[...], w2[...], (((1,), (1,)), ((), ())),
                         preferred_element_type=jnp.float32) + b2[...]
    bc = lax.dot_general(bc, w3[...], (((1,), (1,)), ((), ())),
                         preferred_element_type=jnp.float32) + b3[...]
    pb = p.astype(jnp.bfloat16)
    o_ref[:, :D] = lax.dot_general(
        t0_ref[...].astype(jnp.bfloat16), pb, (((0,), (1,)), ((), ())),
        preferred_element_type=jnp.float32) + bc
    o_ref[:, D:] = lax.dot_general(
        t1_ref[...].astype(jnp.bfloat16), pb, (((0,), (1,)), ((), ())),
        preferred_element_type=jnp.float32) + bc


def _tc_transform(tableT, w1, w2, w3, b1, b2, b3):
    wspec = pl.BlockSpec((D, D), lambda c: (0, 0))
    bspec = pl.BlockSpec((1, D), lambda c: (0, 0))
    return pl.pallas_call(
        _transform_body,
        grid=(GRID_A,),
        in_specs=[pl.BlockSpec((D, BS), lambda c: (0, 2 * c)),
                  pl.BlockSpec((D, BS),
                               lambda c: (0, jnp.minimum(2 * c + 1,
                                                         2 * GRID_A - 2))),
                  wspec, wspec, wspec, bspec, bspec, bspec],
        out_specs=pl.BlockSpec((BS, 2 * D), lambda c: (c, 0)),
        out_shape=jax.ShapeDtypeStruct((GRID_A * BS, 2 * D), jnp.float32),
    )(tableT, tableT, w1, w2, w3, b1, b2, b3)


def _gather_body(table_hbm, idx_hbm, out_hbm, idx_v, rows_v, sem):
    wid = lax.axis_index("s") * NUM_CORES + lax.axis_index("c")
    nbase = wid * ROWS_PER_W
    pltpu.sync_copy(idx_hbm.at[pl.ds(wid * IDX_ROWS_PER_W, IDX_ROWS_PER_W)],
                    idx_v)

    def remap_body(t, carry):
        row = t // 8
        g = (t % 8) * 16
        i = idx_v[row, pl.ds(g, 16)]
        r = ((i >> 15) << 15) + ((i & 16383) << 1) + ((i >> 14) & 1)
        idx_v[row, pl.ds(g, 16)] = r
        return carry

    lax.fori_loop(0, REMAP_VECS, remap_body, 0)

    def chunk_body(t, carry):
        copies = []
        for j in range(CHUNK_STREAMS):
            copies.append(pltpu.async_copy(
                table_hbm.at[idx_v.at[t * CHUNK_STREAMS + j]],
                rows_v.at[pl.ds(j * STREAM_ROWS, STREAM_ROWS)],
                sem))
        for c in copies:
            c.wait()
        n0 = nbase + t * CHUNK_ROWS
        q0 = ((n0 >> 12) << 11) + (n0 & 2047)
        k = (n0 >> 11) & 1
        pltpu.sync_copy(rows_v,
                        out_hbm.at[pl.ds(q0, CHUNK_ROWS),
                                   pl.ds(k * D, D)])
        return carry

    lax.fori_loop(0, N_CHUNKS, chunk_body, 0)


def _sc_gather(table_flat, idx2d):
    k = pl.kernel(
        _gather_body,
        out_type=jax.ShapeDtypeStruct((HALF, 2 * D), jnp.float32),
        mesh=plsc.VectorSubcoreMesh(core_axis_name="c", subcore_axis_name="s"),
        scratch_types=[
            pltpu.VMEM((IDX_ROWS_PER_W, STREAM_ROWS), jnp.int32),
            pltpu.VMEM((CHUNK_ROWS, D), jnp.float32),
            pltpu.SemaphoreType.DMA,
        ],
        compiler_params=pltpu.CompilerParams(use_tc_tiling_on_sc=False),
    )
    return k(table_flat, idx2d)


def _transpose_body(x_ref, o_ref):
    eye = jnp.float32(1.0) * (lax.broadcasted_iota(jnp.int32, (D, D), 0) ==
                              lax.broadcasted_iota(jnp.int32, (D, D), 1))
    for s in range(BQ // 2048):
        for k in range(2):
            o_ref[:, (2 * s + k) * 2048:(2 * s + k + 1) * 2048] = (
                lax.dot_general(eye,
                                x_ref[s * 2048:(s + 1) * 2048,
                                      k * D:(k + 1) * D],
                                (((1,), (1,)), ((), ())),
                                preferred_element_type=jnp.float32))


def _tc_transpose(xp):
    return pl.pallas_call(
        _transpose_body,
        grid=(HALF // BQ,),
        in_specs=[pl.BlockSpec((BQ, 2 * D), lambda c: (c, 0))],
        out_specs=pl.BlockSpec((D, 2 * BQ), lambda c: (0, c)),
        out_shape=jax.ShapeDtypeStruct((D, B), jnp.float32),
    )(xp)


def kernel(features_values, table, W1, b1, W2, b2, W3, b3):
    idx2d = features_values.astype(jnp.int32).reshape(B // STREAM_ROWS,
                                                      STREAM_ROWS)
    tableT = table.T
    twc = _tc_transform(tableT, W1, W2, W3,
                        b1.reshape(1, D), b2.reshape(1, D), b3.reshape(1, D))
    twc_flat = twc.reshape(VPAD, D)
    emb = _sc_gather(twc_flat, idx2d)
    return _tc_transpose(emb).T
